# baseline (speedup 1.0000x reference)
import functools

import jax
import jax.numpy as jnp
from jax import lax
from jax.experimental import pallas as pl
from jax.experimental.pallas import tpu as pltpu
from jax.experimental.pallas import tpu_sc as plsc

DIM = 768
EPS = 1e-12
RPS = 64
SB = 512


@functools.lru_cache(maxsize=None)
def _build_gather(B, S):
    info = plsc.get_sparse_core_info()
    NW = info.num_cores * info.num_subcores
    rows_per_w = B * S // NW
    n_steps = rows_per_w // RPS
    mesh = plsc.VectorSubcoreMesh(core_axis_name="c", subcore_axis_name="s")

    @functools.partial(
        pl.kernel,
        mesh=mesh,
        out_type=jax.ShapeDtypeStruct((B * S, DIM), jnp.float32),
        scratch_types=[
            pltpu.VMEM((rows_per_w,), jnp.int32),
            pltpu.VMEM((RPS, DIM), jnp.float32),
            pltpu.VMEM((RPS, DIM), jnp.float32),
            pltpu.SemaphoreType.DMA,
            pltpu.SemaphoreType.DMA,
            pltpu.SemaphoreType.DMA,
            pltpu.SemaphoreType.DMA,
        ],
    )
    def body(ids_hbm, word_hbm, g_hbm, idx_all, rows0, rows1, gs0, gs1, ws0, ws1):
        rows = (rows0, rows1)
        gsem = (gs0, gs1)
        wsem = (ws0, ws1)
        wid = lax.axis_index("s") * info.num_cores + lax.axis_index("c")
        row_base = wid * rows_per_w
        pltpu.sync_copy(ids_hbm.at[pl.ds(row_base, rows_per_w)], idx_all)

        def issue(k, g):
            idx = idx_all.at[pl.ds(g * RPS, RPS)]
            pltpu.make_async_copy(word_hbm.at[idx], rows[k], gsem[k]).start()

        def wait_in(k, g):
            idx = idx_all.at[pl.ds(g * RPS, RPS)]
            pltpu.make_async_copy(word_hbm.at[idx], rows[k], gsem[k]).wait()

        def writeout(k, g):
            pltpu.make_async_copy(
                rows[k], g_hbm.at[pl.ds(row_base + g * RPS, RPS)], wsem[k]
            ).start()

        def drain_write(k, g):
            pltpu.make_async_copy(
                rows[k], g_hbm.at[pl.ds(row_base + g * RPS, RPS)], wsem[k]
            ).wait()

        issue(0, 0)

        def pair(go, _):
            g0 = go * 2

            @pl.when(go >= 1)
            def _():
                drain_write(1, g0 - 1)

            issue(1, g0 + 1)
            wait_in(0, g0)
            writeout(0, g0)

            @pl.when(go < n_steps // 2 - 1)
            def _():
                drain_write(0, g0)
                issue(0, g0 + 2)

            wait_in(1, g0 + 1)
            writeout(1, g0 + 1)
            return 0

        lax.fori_loop(0, n_steps // 2, pair, 0)
        drain_write(0, n_steps - 2)
        drain_write(1, n_steps - 1)

    return body


def _ln_block(g_ref, pos_ref, gamma_ref, beta_ref, out_ref):
    x = g_ref[...] + pos_ref[...][None]
    mean = jnp.mean(x, axis=-1, keepdims=True)
    cen = x - mean
    var = jnp.mean(cen * cen, axis=-1, keepdims=True)
    normed = cen * lax.rsqrt(var + EPS)
    out_ref[...] = normed * gamma_ref[...] + beta_ref[...]


@functools.lru_cache(maxsize=None)
def _build_ln(B, S):
    return pl.pallas_call(
        _ln_block,
        grid=(B, S // SB),
        in_specs=[
            pl.BlockSpec((1, SB, DIM), lambda b, s: (b, s, 0)),
            pl.BlockSpec((SB, DIM), lambda b, s: (s, 0)),
            pl.BlockSpec((DIM,), lambda b, s: (0,)),
            pl.BlockSpec((DIM,), lambda b, s: (0,)),
        ],
        out_specs=pl.BlockSpec((1, SB, DIM), lambda b, s: (b, s, 0)),
        out_shape=jax.ShapeDtypeStruct((B, S, DIM), jnp.float32),
    )


def kernel(input_ids, word_table, pos_table, gamma, beta):
    ids = input_ids.astype(jnp.int32)
    B, S = ids.shape
    g = _build_gather(B, S)(ids.reshape(-1), word_table)
    return _build_ln(B, S)(
        g.reshape(B, S, DIM), pos_table[:S], gamma, beta
    )


# speedup vs baseline: 1.0118x; 1.0118x over previous
import functools

import jax
import jax.numpy as jnp
from jax import lax
from jax.experimental import pallas as pl
from jax.experimental.pallas import tpu as pltpu
from jax.experimental.pallas import tpu_sc as plsc

DIM = 768
EPS = 1e-12
RPS = 64
SB = 512


@functools.lru_cache(maxsize=None)
def _build_gather(B, S):
    info = plsc.get_sparse_core_info()
    NW = info.num_cores * info.num_subcores
    rows_per_w = B * S // NW
    n_steps = rows_per_w // RPS
    mesh = plsc.VectorSubcoreMesh(core_axis_name="c", subcore_axis_name="s")

    @functools.partial(
        pl.kernel,
        mesh=mesh,
        out_type=jax.ShapeDtypeStruct((B * S, DIM), jnp.float32),
        scratch_types=[
            pltpu.VMEM((rows_per_w,), jnp.int32),
            pltpu.VMEM((RPS, DIM), jnp.float32),
            pltpu.VMEM((RPS, DIM), jnp.float32),
            pltpu.SemaphoreType.DMA,
            pltpu.SemaphoreType.DMA,
            pltpu.SemaphoreType.DMA,
            pltpu.SemaphoreType.DMA,
        ],
    )
    def body(ids_hbm, word_hbm, g_hbm, idx_all, rows0, rows1, gs0, gs1, ws0, ws1):
        rows = (rows0, rows1)
        gsem = (gs0, gs1)
        wsem = (ws0, ws1)
        wid = lax.axis_index("s") * info.num_cores + lax.axis_index("c")
        row_base = wid * rows_per_w
        pltpu.sync_copy(ids_hbm.at[pl.ds(row_base, rows_per_w)], idx_all)

        def issue(k, g):
            idx = idx_all.at[pl.ds(g * RPS, RPS)]
            pltpu.make_async_copy(word_hbm.at[idx], rows[k], gsem[k]).start()

        def wait_in(k, g):
            idx = idx_all.at[pl.ds(g * RPS, RPS)]
            pltpu.make_async_copy(word_hbm.at[idx], rows[k], gsem[k]).wait()

        def writeout(k, g):
            pltpu.make_async_copy(
                rows[k], g_hbm.at[pl.ds(row_base + g * RPS, RPS)], wsem[k]
            ).start()

        def drain_write(k, g):
            pltpu.make_async_copy(
                rows[k], g_hbm.at[pl.ds(row_base + g * RPS, RPS)], wsem[k]
            ).wait()

        issue(0, 0)

        def pair(go, _):
            g0 = go * 2

            @pl.when(go >= 1)
            def _():
                drain_write(1, g0 - 1)

            issue(1, g0 + 1)
            wait_in(0, g0)
            writeout(0, g0)

            @pl.when(go < n_steps // 2 - 1)
            def _():
                drain_write(0, g0)
                issue(0, g0 + 2)

            wait_in(1, g0 + 1)
            writeout(1, g0 + 1)
            return 0

        lax.fori_loop(0, n_steps // 2, pair, 0)
        drain_write(0, n_steps - 2)
        drain_write(1, n_steps - 1)

    return body


def _ln_block(g_ref, pos_ref, gamma_ref, beta_ref, out_ref):
    x = g_ref[...] + pos_ref[...][None]
    mean = jnp.mean(x, axis=-1, keepdims=True)
    cen = x - mean
    var = jnp.mean(cen * cen, axis=-1, keepdims=True)
    normed = cen * lax.rsqrt(var + EPS)
    out_ref[...] = normed * gamma_ref[...] + beta_ref[...]


@functools.lru_cache(maxsize=None)
def _build_ln(B, S):
    return pl.pallas_call(
        _ln_block,
        grid=(S // SB, B),
        in_specs=[
            pl.BlockSpec((1, SB, DIM), lambda s, b: (b, s, 0)),
            pl.BlockSpec((SB, DIM), lambda s, b: (s, 0)),
            pl.BlockSpec((DIM,), lambda s, b: (0,)),
            pl.BlockSpec((DIM,), lambda s, b: (0,)),
        ],
        out_specs=pl.BlockSpec((1, SB, DIM), lambda s, b: (b, s, 0)),
        out_shape=jax.ShapeDtypeStruct((B, S, DIM), jnp.float32),
    )


def kernel(input_ids, word_table, pos_table, gamma, beta):
    ids = input_ids.astype(jnp.int32)
    B, S = ids.shape
    g = _build_gather(B, S)(ids.reshape(-1), word_table)
    return _build_ln(B, S)(
        g.reshape(B, S, DIM), pos_table[:S], gamma, beta
    )
